# baseline (speedup 1.0000x reference)
import functools

import jax
import jax.numpy as jnp
from jax import lax
from jax.experimental import pallas as pl
from jax.experimental.pallas import tpu as pltpu
from jax.experimental.pallas import tpu_sc as plsc

_NC = 2
_NS = 16
_NW = _NC * _NS
_L = 16

_C = 40
_NBUF = 4
_BR = 40


def _sc_gather_scatter(y, colp, rowp, evp, n, d, nchunk):
    nrounds = (nchunk - 2) // _NBUF
    npeel = nchunk - nrounds * _NBUF
    nblk = n // _BR

    mesh = plsc.VectorSubcoreMesh(core_axis_name="c", subcore_axis_name="s")

    @functools.partial(
        pl.kernel,
        out_type=jax.ShapeDtypeStruct((_NC, n, d), jnp.float32),
        mesh=mesh,
        compiler_params=pltpu.CompilerParams(needs_layout_passes=False),
        scratch_types=[
            pltpu.VMEM((_NBUF, _C), jnp.int32),
            pltpu.VMEM((_NBUF, _C), jnp.int32),
            pltpu.VMEM((_NBUF * _C,), jnp.float32),
            pltpu.VMEM((_NBUF, _C, d), jnp.float32),
            pltpu.VMEM_SHARED((n, d), jnp.float32),
            pltpu.SemaphoreType.DMA((_NBUF,)),
            pltpu.SemaphoreType.DMA((_NBUF,)),
            pltpu.SemaphoreType.DMA((_NBUF,)),
            pltpu.SemaphoreType.DMA,
        ],
    )
    def sc_kernel(y_hbm, col_hbm, row_hbm, ev_hbm, out_hbm,
                  colb, rowb, evb, msgs_v, agg_s, isem, gsem, ssem, bsem):
        c = lax.axis_index("c")
        s = lax.axis_index("s")
        w = s * _NC + c

        def zero_body(e, carry):
            for v in range(d // _L):
                msgs_v[0, e, pl.ds(v * _L, _L)] = jnp.zeros((_L,), jnp.float32)
            return carry
        lax.fori_loop(0, _BR, zero_body, 0)
        for k in range(-(-nblk // _NS)):
            blk = k * _NS + s

            @pl.when(blk < nblk)
            def _():
                pltpu.async_copy(msgs_v.at[0, pl.ds(0, _BR)],
                                 agg_s.at[pl.ds(blk * _BR, _BR)], bsem)
        for k in range(-(-nblk // _NS)):
            blk = k * _NS + s

            @pl.when(blk < nblk)
            def _():
                pltpu.make_async_copy(msgs_v.at[0, pl.ds(0, _BR)],
                                      agg_s.at[pl.ds(blk * _BR, _BR)],
                                      bsem).wait()
        plsc.subcore_barrier()

        def idx_copies(chunk, buf):
            base = (w * nchunk + chunk) * _C
            return (
                pltpu.make_async_copy(col_hbm.at[pl.ds(base, _C)],
                                      colb.at[buf], isem.at[buf]),
                pltpu.make_async_copy(row_hbm.at[pl.ds(base, _C)],
                                      rowb.at[buf], isem.at[buf]),
                pltpu.make_async_copy(ev_hbm.at[pl.ds(base, _C)],
                                      evb.at[pl.ds(buf * _C, _C)],
                                      isem.at[buf]),
            )

        def start_idx(chunk, buf):
            for cp in idx_copies(chunk, buf):
                cp.start()

        def wait_idx(chunk, buf):
            for cp in idx_copies(chunk, buf):
                cp.wait()

        def start_gather(buf):
            pltpu.async_copy(y_hbm.at[colb.at[buf]], msgs_v.at[buf],
                             gsem.at[buf])

        def wait_gather(buf):
            pltpu.make_async_copy(y_hbm.at[colb.at[buf]], msgs_v.at[buf],
                                  gsem.at[buf]).wait()

        def scale(buf):
            @plsc.parallel_loop(0, _C, unroll=4)
            def _(e):
                idx = jnp.full((_L,), buf * _C, jnp.int32) + e
                evs = plsc.load_gather(evb, [idx])
                for v in range(d // _L):
                    cur = msgs_v[buf, e, pl.ds(v * _L, _L)]
                    msgs_v[buf, e, pl.ds(v * _L, _L)] = cur * evs

        def start_scatter(buf):
            pltpu.async_copy(msgs_v.at[buf], agg_s.at[rowb.at[buf]],
                             ssem.at[buf], add=True)

        def wait_scatter(buf):
            pltpu.make_async_copy(msgs_v.at[buf], agg_s.at[rowb.at[buf]],
                                  ssem.at[buf]).wait()

        def chunk_step(i, b, static):
            def when(cond, fn):
                if static:
                    if cond:
                        fn()
                else:
                    pl.when(cond)(fn)

            when(i >= 2, lambda: wait_scatter((b + 2) % _NBUF))
            when(i + 2 < nchunk, lambda: start_idx(i + 2, (b + 2) % _NBUF))

            def adv():
                wait_idx(i + 1, (b + 1) % _NBUF)
                start_gather((b + 1) % _NBUF)
            when(i + 1 < nchunk, adv)

            wait_gather(b)
            scale(b)
            start_scatter(b)

        start_idx(0, 0)
        start_idx(1, 1)
        wait_idx(0, 0)
        start_gather(0)

        def round_body(r, carry):
            for b in range(_NBUF):
                chunk_step(r * _NBUF + b, b, static=False)
            return carry

        lax.fori_loop(0, nrounds, round_body, 0)
        for t in range(npeel):
            i = nrounds * _NBUF + t
            chunk_step(i, i % _NBUF, static=True)
        wait_scatter((nchunk - 2) % _NBUF)
        wait_scatter((nchunk - 1) % _NBUF)

        plsc.subcore_barrier()
        for k in range(-(-nblk // _NS)):
            blk = k * _NS + s

            @pl.when(blk < nblk)
            def _():
                pltpu.async_copy(agg_s.at[pl.ds(blk * _BR, _BR)],
                                 out_hbm.at[c, pl.ds(blk * _BR, _BR)], bsem)
        for k in range(-(-nblk // _NS)):
            blk = k * _NS + s

            @pl.when(blk < nblk)
            def _():
                pltpu.make_async_copy(agg_s.at[pl.ds(blk * _BR, _BR)],
                                      out_hbm.at[c, pl.ds(blk * _BR, _BR)],
                                      bsem).wait()

    return sc_kernel(y, colp, rowp, evp)


def _tc_project(x, W, n, d):
    mb = 2000

    def body(x_ref, w_ref, o_ref):
        o_ref[...] = jnp.dot(x_ref[...], w_ref[...],
                             preferred_element_type=jnp.float32)

    return pl.pallas_call(
        body,
        grid=(n // mb,),
        in_specs=[
            pl.BlockSpec((mb, d), lambda i: (i, 0)),
            pl.BlockSpec((d, d), lambda i: (0, 0)),
        ],
        out_specs=pl.BlockSpec((mb, d), lambda i: (i, 0)),
        out_shape=jax.ShapeDtypeStruct((n, d), jnp.float32),
    )(x, W)


def _tc_combine(partial, b2, n, d):
    mb = 2000

    def body(p_ref, b_ref, o_ref):
        o_ref[...] = p_ref[0] + p_ref[1] + b_ref[...]

    return pl.pallas_call(
        body,
        grid=(n // mb,),
        in_specs=[
            pl.BlockSpec((_NC, mb, d), lambda i: (0, i, 0)),
            pl.BlockSpec((1, d), lambda i: (0, 0)),
        ],
        out_specs=pl.BlockSpec((mb, d), lambda i: (i, 0)),
        out_shape=jax.ShapeDtypeStruct((n, d), jnp.float32),
    )(partial, b2)


def kernel(x, edge_index, edge_vals, W, b):
    n, d_in = x.shape
    d_out = W.shape[1]
    e = edge_index.shape[1]

    ep = e // _NW
    nchunk = ep // _C

    row = edge_index[0]
    col = edge_index[1]

    y = _tc_project(x, W, n, d_in)
    partial = _sc_gather_scatter(y, col, row, edge_vals, n, d_out, nchunk)
    return _tc_combine(partial, b.reshape(1, d_out), n, d_out)


# speedup vs baseline: 1.2914x; 1.2914x over previous
import functools

import jax
import jax.numpy as jnp
from jax import lax
from jax.experimental import pallas as pl
from jax.experimental.pallas import tpu as pltpu
from jax.experimental.pallas import tpu_sc as plsc

_NC = 2
_NS = 16
_NW = _NC * _NS
_L = 16

_C = 40
_NBUF = 4
_NIDX = 8
_RU = 8
_BR = 40


def _sc_gather_scatter(x, colp, rowp, evp, n, d, nchunk):
    nrounds = (nchunk - 2) // _RU
    npeel = nchunk - nrounds * _RU
    nblk = n // _BR

    mesh = plsc.VectorSubcoreMesh(core_axis_name="c", subcore_axis_name="s")

    @functools.partial(
        pl.kernel,
        out_type=jax.ShapeDtypeStruct((_NC, n, d), jnp.float32),
        mesh=mesh,
        compiler_params=pltpu.CompilerParams(needs_layout_passes=False),
        scratch_types=[
            pltpu.VMEM((_NIDX, _C), jnp.int32),
            pltpu.VMEM((_NIDX, _C), jnp.int32),
            pltpu.VMEM((_NIDX * _C,), jnp.float32),
            pltpu.VMEM((_NBUF, _C, d), jnp.float32),
            pltpu.VMEM_SHARED((n, d), jnp.float32),
            pltpu.SemaphoreType.DMA((_NIDX,)),
            pltpu.SemaphoreType.DMA((_NBUF,)),
            pltpu.SemaphoreType.DMA((_NBUF,)),
            pltpu.SemaphoreType.DMA,
        ],
    )
    def sc_kernel(x_hbm, col_hbm, row_hbm, ev_hbm, out_hbm,
                  colb, rowb, evb, msgs_v, agg_s, isem, gsem, ssem, bsem):
        c = lax.axis_index("c")
        s = lax.axis_index("s")
        w = s * _NC + c

        def idx_copies(chunk, q):
            base = (w * nchunk + chunk) * _C
            return (
                pltpu.make_async_copy(col_hbm.at[pl.ds(base, _C)],
                                      colb.at[q], isem.at[q]),
                pltpu.make_async_copy(row_hbm.at[pl.ds(base, _C)],
                                      rowb.at[q], isem.at[q]),
                pltpu.make_async_copy(ev_hbm.at[pl.ds(base, _C)],
                                      evb.at[pl.ds(q * _C, _C)],
                                      isem.at[q]),
            )

        def start_idx(chunk, q):
            for cp in idx_copies(chunk, q):
                cp.start()

        def wait_idx(chunk, q):
            for cp in idx_copies(chunk, q):
                cp.wait()

        def start_gather(b, q):
            pltpu.async_copy(x_hbm.at[colb.at[q]], msgs_v.at[b], gsem.at[b])

        def wait_gather(b, q):
            pltpu.make_async_copy(x_hbm.at[colb.at[q]], msgs_v.at[b],
                                  gsem.at[b]).wait()

        def scale(b, q):
            @plsc.parallel_loop(0, _C, unroll=4)
            def _(e):
                idx = jnp.full((_L,), q * _C, jnp.int32) + e
                evs = plsc.load_gather(evb, [idx])
                for v in range(d // _L):
                    cur = msgs_v[b, e, pl.ds(v * _L, _L)]
                    msgs_v[b, e, pl.ds(v * _L, _L)] = cur * evs

        def start_scatter(b, q):
            pltpu.async_copy(msgs_v.at[b], agg_s.at[rowb.at[q]],
                             ssem.at[b], add=True)

        def wait_scatter(b, q):
            pltpu.make_async_copy(msgs_v.at[b], agg_s.at[rowb.at[q]],
                                  ssem.at[b]).wait()

        for j in range(4):
            start_idx(j, j)

        def zero_body(e, carry):
            for v in range(d // _L):
                msgs_v[0, e, pl.ds(v * _L, _L)] = jnp.zeros((_L,), jnp.float32)
            return carry
        lax.fori_loop(0, _BR, zero_body, 0)
        for k in range(-(-nblk // _NS)):
            blk = k * _NS + s

            @pl.when(blk < nblk)
            def _():
                pltpu.async_copy(msgs_v.at[0, pl.ds(0, _BR)],
                                 agg_s.at[pl.ds(blk * _BR, _BR)], bsem)
        for k in range(-(-nblk // _NS)):
            blk = k * _NS + s

            @pl.when(blk < nblk)
            def _():
                pltpu.make_async_copy(msgs_v.at[0, pl.ds(0, _BR)],
                                      agg_s.at[pl.ds(blk * _BR, _BR)],
                                      bsem).wait()

        wait_idx(0, 0)
        start_gather(0, 0)
        wait_idx(1, 1)
        start_gather(1, 1)
        plsc.subcore_barrier()

        def chunk_step(i, b, q, static):
            def when(cond, fn):
                if static:
                    if cond:
                        fn()
                else:
                    pl.when(cond)(fn)

            when(i >= 2, lambda: wait_scatter((b + 2) % _NBUF,
                                              (q + 6) % _NIDX))
            when(i + 4 < nchunk, lambda: start_idx(i + 4, (q + 4) % _NIDX))

            def adv():
                wait_idx(i + 2, (q + 2) % _NIDX)
                start_gather((b + 2) % _NBUF, (q + 2) % _NIDX)
            when(i + 2 < nchunk, adv)

            wait_gather(b, q)
            scale(b, q)
            start_scatter(b, q)

        def round_body(r, carry):
            for u in range(_RU):
                chunk_step(r * _RU + u, u % _NBUF, u % _NIDX, static=False)
            return carry

        lax.fori_loop(0, nrounds, round_body, 0)
        for t in range(npeel):
            i = nrounds * _RU + t
            chunk_step(i, t % _NBUF, t % _NIDX, static=True)
        wait_scatter((nchunk - 2) % _NBUF, (nchunk - 2) % _NIDX)
        wait_scatter((nchunk - 1) % _NBUF, (nchunk - 1) % _NIDX)

        plsc.subcore_barrier()
        for k in range(-(-nblk // _NS)):
            blk = k * _NS + s

            @pl.when(blk < nblk)
            def _():
                pltpu.async_copy(agg_s.at[pl.ds(blk * _BR, _BR)],
                                 out_hbm.at[c, pl.ds(blk * _BR, _BR)], bsem)
        for k in range(-(-nblk // _NS)):
            blk = k * _NS + s

            @pl.when(blk < nblk)
            def _():
                pltpu.make_async_copy(agg_s.at[pl.ds(blk * _BR, _BR)],
                                      out_hbm.at[c, pl.ds(blk * _BR, _BR)],
                                      bsem).wait()

    return sc_kernel(x, colp, rowp, evp)


def _tc_combine_matmul(partial, W, b2, n, d):
    mb = 2000

    def body(p_ref, w_ref, b_ref, o_ref):
        agg = p_ref[0] + p_ref[1]
        o_ref[...] = (
            jnp.dot(agg, w_ref[...], preferred_element_type=jnp.float32)
            + b_ref[...]
        )

    return pl.pallas_call(
        body,
        grid=(n // mb,),
        in_specs=[
            pl.BlockSpec((_NC, mb, d), lambda i: (0, i, 0)),
            pl.BlockSpec((d, d), lambda i: (0, 0)),
            pl.BlockSpec((1, d), lambda i: (0, 0)),
        ],
        out_specs=pl.BlockSpec((mb, d), lambda i: (i, 0)),
        out_shape=jax.ShapeDtypeStruct((n, d), jnp.float32),
    )(partial, W, b2)


def kernel(x, edge_index, edge_vals, W, b):
    n, d_in = x.shape
    d_out = W.shape[1]
    e = edge_index.shape[1]

    ep = e // _NW
    nchunk = ep // _C

    row = edge_index[0]
    col = edge_index[1]

    partial = _sc_gather_scatter(x, col, row, edge_vals, n, d_in, nchunk)
    return _tc_combine_matmul(partial, W, b.reshape(1, d_out), n, d_out)
